# baseline (speedup 1.0000x reference)
import functools

import jax
import jax.numpy as jnp
from jax import lax
from jax.experimental import pallas as pl
from jax.experimental.pallas import tpu as pltpu
from jax.experimental.pallas import tpu_sc as plsc

BUFFER_SIZE = 10000
N_CLASSES = 100
BATCH = 4096
ROW = 3 * 32 * 32

R1 = BATCH * ROW
R2 = (BUFFER_SIZE - BATCH) * ROW
TOT = BUFFER_SIZE * ROW
TAIL = BUFFER_SIZE - BATCH

CHUNK_ROWS = 64
NB = 12
DEPTH = 6


NSPAN = 16


def _bx_direct_body(x_h, bx_h, obx_h, sems):
    ds = []
    for k in range(NSPAN):
        off = k * (BATCH // NSPAN)
        ds.append(pltpu.make_async_copy(
            x_h.at[pl.ds(off, BATCH // NSPAN)],
            obx_h.at[pl.ds(off, BATCH // NSPAN)], sems.at[k]))
    for k in range(NSPAN):
        off = BATCH + k * (TAIL // NSPAN)
        ds.append(pltpu.make_async_copy(
            bx_h.at[pl.ds(off, TAIL // NSPAN)],
            obx_h.at[pl.ds(off, TAIL // NSPAN)], sems.at[NSPAN + k]))
    for d in ds:
        d.start()
    for d in ds:
        d.wait()


def _bx_body(x_h, bx_h, obx_h, *bufs_and_sems):
    bufs = bufs_and_sems[:NB]
    sin = bufs_and_sems[NB]
    sout = bufs_and_sems[NB + 1]

    chunks = []
    for off in range(0, BATCH, CHUNK_ROWS):
        chunks.append((x_h, off, min(CHUNK_ROWS, BATCH - off)))
    for off in range(BATCH, BUFFER_SIZE, CHUNK_ROWS):
        chunks.append((bx_h, off, min(CHUNK_ROWS, BUFFER_SIZE - off)))
    n = len(chunks)

    in_d = [None] * n
    out_d = [None] * n

    def start_in(i):
        src, off, sz = chunks[i]
        b = i % NB
        in_d[i] = pltpu.make_async_copy(
            src.at[pl.ds(off, sz)], bufs[b].at[pl.ds(0, sz)], sin.at[b])
        in_d[i].start()

    for i in range(min(DEPTH, n)):
        start_in(i)
    for i in range(n):
        _, off, sz = chunks[i]
        b = i % NB
        in_d[i].wait()
        out_d[i] = pltpu.make_async_copy(
            bufs[b].at[pl.ds(0, sz)], obx_h.at[pl.ds(off, sz)], sout.at[b])
        out_d[i].start()
        j = i + DEPTH
        if j < n:
            if j >= NB:
                out_d[j - NB].wait()
            start_in(j)
    for i in range(max(0, n - NB), n):
        out_d[i].wait()


def _sc_body(y_h, t_h, by_h, bt_h, oby_h, obt_h, stage, tvec, sem0, sem1):
    cid = lax.axis_index("c")
    sid = lax.axis_index("s")
    wid = sid * 2 + cid

    @pl.when(wid == 0)
    def _():
        d0 = pltpu.async_copy(y_h, stage.at[pl.ds(0, BATCH)], sem0)
        d1 = pltpu.async_copy(by_h.at[pl.ds(BATCH, TAIL)],
                              stage.at[pl.ds(BATCH, TAIL)], sem1)
        d0.wait()
        pltpu.async_copy(stage.at[pl.ds(0, BATCH)],
                         oby_h.at[pl.ds(0, BATCH)], sem0).wait()
        d1.wait()
        pltpu.async_copy(stage.at[pl.ds(BATCH, TAIL)],
                         oby_h.at[pl.ds(BATCH, TAIL)], sem1).wait()

    @pl.when(wid == 1)
    def _():
        pltpu.sync_copy(t_h, tvec)
        d1 = pltpu.async_copy(bt_h.at[pl.ds(BATCH, TAIL)],
                              stage.at[pl.ds(BATCH, TAIL)], sem1)
        tv = tvec[...]
        for i in range(BATCH // 16):
            stage[pl.ds(i * 16, 16)] = tv
        pltpu.async_copy(stage.at[pl.ds(0, BATCH)],
                         obt_h.at[pl.ds(0, BATCH)], sem0).wait()
        d1.wait()
        pltpu.async_copy(stage.at[pl.ds(BATCH, TAIL)],
                         obt_h.at[pl.ds(BATCH, TAIL)], sem1).wait()


@jax.jit
def _er_update(x, y, t16, bx, by, bt):
    obx = pl.pallas_call(
        _bx_direct_body,
        in_specs=[pl.BlockSpec(memory_space=pltpu.MemorySpace.HBM),
                  pl.BlockSpec(memory_space=pltpu.MemorySpace.HBM)],
        out_specs=pl.BlockSpec(memory_space=pltpu.MemorySpace.HBM),
        out_shape=jax.ShapeDtypeStruct(bx.shape, jnp.float32),
        scratch_shapes=[pltpu.SemaphoreType.DMA((2 * NSPAN,))],
    )(x, bx)

    mesh = plsc.VectorSubcoreMesh(core_axis_name="c", subcore_axis_name="s")
    oby, obt = pl.kernel(
        _sc_body,
        out_type=(
            jax.ShapeDtypeStruct((BUFFER_SIZE,), jnp.int32),
            jax.ShapeDtypeStruct((BUFFER_SIZE,), jnp.int32),
        ),
        mesh=mesh,
        scratch_types=[
            pltpu.VMEM((BUFFER_SIZE,), jnp.int32),
            pltpu.VMEM((16,), jnp.int32),
            pltpu.SemaphoreType.DMA,
            pltpu.SemaphoreType.DMA,
        ],
    )(y, t16, by, bt)
    return obx, oby, obt


def kernel(x, y, task_id, bx, by, bt):
    t16 = jnp.full((16,), task_id, dtype=jnp.int32)
    return _er_update(x, y, t16, bx, by, bt)


# speedup vs baseline: 12.8818x; 12.8818x over previous
import functools

import jax
import jax.numpy as jnp
from jax import lax
from jax.experimental import pallas as pl
from jax.experimental.pallas import tpu as pltpu
from jax.experimental.pallas import tpu_sc as plsc

BUFFER_SIZE = 10000
N_CLASSES = 100
BATCH = 4096
TAIL = BUFFER_SIZE - BATCH

NTILES = 32
CR = 4
NB = 2
PT1 = BATCH // NTILES
PT2 = TAIL // NTILES
REM_BASE = BATCH + NTILES * PT2


def _body(x_h, y_h, t_h, bx_h, by_h, bt_h, obx_h, oby_h, obt_h,
          spbuf, tfill, tailb, tvec, sem0, sem1):
    cid = lax.axis_index("c")
    sid = lax.axis_index("s")
    wid = sid * 2 + cid

    sems = (sem0, sem1)

    def copy_rows(chunks):
        n = len(chunks)
        in_d = [None] * n
        out_d = [None] * n
        for i in range(n):
            b = i % NB
            if i >= NB:
                out_d[i - NB].wait()
            src, off, nr = chunks[i]
            in_d[i] = pltpu.async_copy(
                src.at[pl.ds(off, nr)],
                spbuf.at[sid, b, pl.ds(0, nr)], sems[b])
            in_d[i].wait()
            out_d[i] = pltpu.async_copy(
                spbuf.at[sid, b, pl.ds(0, nr)],
                obx_h.at[pl.ds(off, nr)], sems[b])
        for i in range(max(0, n - NB), n):
            out_d[i].wait()

    chunks = []
    base1 = wid * PT1
    for k in range(PT1 // CR):
        chunks.append((x_h, base1 + k * CR, CR))
    base2 = BATCH + wid * PT2
    for k in range(PT2 // CR):
        chunks.append((bx_h, base2 + k * CR, CR))
    copy_rows(chunks)

    @pl.when(wid < 4)
    def _():
        off = REM_BASE + wid * CR
        d = pltpu.async_copy(bx_h.at[pl.ds(off, CR)],
                             spbuf.at[sid, 0, pl.ds(0, CR)], sem0)
        d.wait()
        pltpu.async_copy(spbuf.at[sid, 0, pl.ds(0, CR)],
                         obx_h.at[pl.ds(off, CR)], sem0).wait()

    @pl.when(wid == 30)
    def _():
        d0 = pltpu.async_copy(y_h, tfill, sem0)
        d1 = pltpu.async_copy(by_h.at[pl.ds(BATCH, TAIL)], tailb, sem1)
        d0.wait()
        pltpu.async_copy(tfill, oby_h.at[pl.ds(0, BATCH)], sem0).wait()
        d1.wait()
        pltpu.async_copy(tailb, oby_h.at[pl.ds(BATCH, TAIL)], sem1).wait()

    @pl.when(wid == 31)
    def _():
        pltpu.sync_copy(t_h, tvec)
        d1 = pltpu.async_copy(bt_h.at[pl.ds(BATCH, TAIL)], tailb, sem1)
        tv = tvec[...]
        for i in range(BATCH // 16):
            tfill[pl.ds(i * 16, 16)] = tv
        pltpu.async_copy(tfill, obt_h.at[pl.ds(0, BATCH)], sem0).wait()
        d1.wait()
        pltpu.async_copy(tailb, obt_h.at[pl.ds(BATCH, TAIL)], sem1).wait()


@jax.jit
def _er_update(x, y, t16, bx, by, bt):
    mesh = plsc.VectorSubcoreMesh(core_axis_name="c", subcore_axis_name="s")
    run = pl.kernel(
        _body,
        out_type=(
            jax.ShapeDtypeStruct(bx.shape, jnp.float32),
            jax.ShapeDtypeStruct((BUFFER_SIZE,), jnp.int32),
            jax.ShapeDtypeStruct((BUFFER_SIZE,), jnp.int32),
        ),
        mesh=mesh,
        scratch_types=[
            pltpu.VMEM_SHARED((16, NB, CR) + (3, 32, 32), jnp.float32),
            pltpu.VMEM((BATCH,), jnp.int32),
            pltpu.VMEM((TAIL,), jnp.int32),
            pltpu.VMEM((16,), jnp.int32),
            pltpu.SemaphoreType.DMA,
            pltpu.SemaphoreType.DMA,
        ],
        compiler_params=pltpu.CompilerParams(use_tc_tiling_on_sc=True),
    )
    return run(x, y, t16, bx, by, bt)


def kernel(x, y, task_id, bx, by, bt):
    t16 = jnp.full((16,), task_id, dtype=jnp.int32)
    return _er_update(x, y, t16, bx, by, bt)


# speedup vs baseline: 13.9868x; 1.0858x over previous
import functools

import jax
import jax.numpy as jnp
from jax import lax
from jax.experimental import pallas as pl
from jax.experimental.pallas import tpu as pltpu
from jax.experimental.pallas import tpu_sc as plsc

BUFFER_SIZE = 10000
N_CLASSES = 100
BATCH = 4096
TAIL = BUFFER_SIZE - BATCH
ROW = 3 * 32 * 32

R1 = BATCH * ROW
TOT = BUFFER_SIZE * ROW
R2 = TOT - R1

NTILES = 32
S1 = R1 // NTILES
S2 = R2 // NTILES

CHUNK = 49_152


def _body(x_h, y_h, t_h, bx_h, by_h, bt_h, obx_h, oby_h, obt_h,
          spbuf, tfill, tailb, tvec, sem0, sem1):
    cid = lax.axis_index("c")
    sid = lax.axis_index("s")
    wid = sid * 2 + cid

    sems = (sem0, sem1)

    def copy_span(src_h, off0, sizes):
        n = len(sizes)
        offs = [off0]
        for s in sizes[:-1]:
            offs.append(offs[-1] + s)
        in_d = [None] * n
        out_d = [None] * n
        for i in range(n):
            b = i % 2
            if i >= 2:
                out_d[i - 2].wait()
            in_d[i] = pltpu.async_copy(
                src_h.at[pl.ds(offs[i], sizes[i])],
                spbuf.at[sid, b, pl.ds(0, sizes[i])], sems[b])
            in_d[i].wait()
            out_d[i] = pltpu.async_copy(
                spbuf.at[sid, b, pl.ds(0, sizes[i])],
                obx_h.at[pl.ds(offs[i], sizes[i])], sems[b])
        for i in range(max(0, n - 2), n):
            out_d[i].wait()

    copy_span(x_h, wid * S1, [CHUNK] * (S1 // CHUNK))
    n2, rem = divmod(S2, CHUNK)
    copy_span(bx_h, R1 + wid * S2, [CHUNK] * n2 + ([rem] if rem else []))

    @pl.when(wid == 30)
    def _():
        d0 = pltpu.async_copy(y_h, tfill, sem0)
        d1 = pltpu.async_copy(by_h.at[pl.ds(BATCH, TAIL)], tailb, sem1)
        d0.wait()
        pltpu.async_copy(tfill, oby_h.at[pl.ds(0, BATCH)], sem0).wait()
        d1.wait()
        pltpu.async_copy(tailb, oby_h.at[pl.ds(BATCH, TAIL)], sem1).wait()

    @pl.when(wid == 31)
    def _():
        pltpu.sync_copy(t_h, tvec)
        d1 = pltpu.async_copy(bt_h.at[pl.ds(BATCH, TAIL)], tailb, sem1)
        tv = tvec[...]
        for i in range(BATCH // 16):
            tfill[pl.ds(i * 16, 16)] = tv
        pltpu.async_copy(tfill, obt_h.at[pl.ds(0, BATCH)], sem0).wait()
        d1.wait()
        pltpu.async_copy(tailb, obt_h.at[pl.ds(BATCH, TAIL)], sem1).wait()


@jax.jit
def _er_update(x, y, t16, bx, by, bt):
    xf = x.reshape(R1)
    bxf = bx.reshape(TOT)
    mesh = plsc.VectorSubcoreMesh(core_axis_name="c", subcore_axis_name="s")
    obx, oby, obt = pl.kernel(
        _body,
        out_type=(
            jax.ShapeDtypeStruct((TOT,), jnp.float32),
            jax.ShapeDtypeStruct((BUFFER_SIZE,), jnp.int32),
            jax.ShapeDtypeStruct((BUFFER_SIZE,), jnp.int32),
        ),
        mesh=mesh,
        scratch_types=[
            pltpu.VMEM_SHARED((16, 2, CHUNK), jnp.float32),
            pltpu.VMEM((BATCH,), jnp.int32),
            pltpu.VMEM((TAIL,), jnp.int32),
            pltpu.VMEM((16,), jnp.int32),
            pltpu.SemaphoreType.DMA,
            pltpu.SemaphoreType.DMA,
        ],
    )(xf, y, t16, bxf, by, bt)
    return obx.reshape(bx.shape), oby, obt


def kernel(x, y, task_id, bx, by, bt):
    t16 = jnp.full((16,), task_id, dtype=jnp.int32)
    return _er_update(x, y, t16, bx, by, bt)
